# baseline (speedup 1.0000x reference)
import functools

import jax
import jax.numpy as jnp
from jax import lax
from jax.experimental import pallas as pl
from jax.experimental.pallas import tpu as pltpu
from jax.experimental.pallas import tpu_sc as plsc

B, L, H, VOCAB = 128, 512, 768, 8
EPS = 1e-5

NC, NS, LANES = 2, 16, 16
NW = NC * NS
N = B * L


def _table_kernel(ids_ref, tok_ref, tokv_ref, pos_ref, gamma_ref, beta_ref,
                  t_ref, idst_ref, rstd_s, rm_s):
    v = pl.program_id(0)

    @pl.when(v == 0)
    def _():
        idst_ref[...] = ids_ref[...].T
        tok = tok_ref[...]
        pos = pos_ref[...]
        inv_h = 1.0 / H
        ones_row = jnp.ones((1, H), dtype=jnp.float32)
        mp = jnp.mean(pos, axis=1, keepdims=True)
        ep2 = jnp.mean(pos * pos, axis=1, keepdims=True)
        mt = lax.dot_general(ones_row, tok, (((1,), (1,)), ((), ())),
                             preferred_element_type=jnp.float32) * inv_h
        et2 = lax.dot_general(ones_row, tok * tok, (((1,), (1,)), ((), ())),
                              preferred_element_type=jnp.float32) * inv_h
        cross = lax.dot_general(pos, tok, (((1,), (1,)), ((), ())),
                                preferred_element_type=jnp.float32) * inv_h
        mu = mp + mt
        var = ep2 + et2 + 2.0 * cross - mu * mu
        rstd = lax.rsqrt(var + EPS)
        rstd_s[...] = rstd
        rm_s[...] = rstd * mu

    sel = (lax.broadcasted_iota(jnp.int32, (L, VOCAB), 1) == v).astype(jnp.float32)
    a = jnp.sum(rstd_s[...] * sel, axis=1, keepdims=True)
    s = jnp.sum(rm_s[...] * sel, axis=1, keepdims=True)
    t = (pos_ref[...] + tokv_ref[0]) * a - s
    t_ref[:, 0, 0, :] = t * gamma_ref[0][None, :] + beta_ref[0][None, :]


LPW = L // NW


LAG = 4


def _sc_gather(t_hbm, idst_hbm, out_hbm, tl_v, ids_v, sem):
    wid = lax.axis_index("s") * NC + lax.axis_index("c")
    l0 = wid * LPW
    pltpu.sync_copy(t_hbm.at[pl.ds(wid * LPW * VOCAB, LPW * VOCAB)], tl_v)
    pltpu.sync_copy(idst_hbm.at[pl.ds(l0 * B, LPW * B)], ids_v)

    lanesb = lax.iota(jnp.int32, LANES) * B

    def fire(b):
        v = plsc.load_gather(ids_v, [lanesb + b])
        for l in range(LPW):
            r = v[l] + l * VOCAB
            pltpu.async_copy(
                tl_v.at[pl.ds(r, 1)], out_hbm.at[pl.ds(b * L + l0 + l, 1)], sem
            )

    def drain_one_batch():
        pltpu.make_async_copy(
            tl_v.at[pl.ds(0, LPW)], out_hbm.at[pl.ds(l0, LPW)], sem
        ).wait()

    for b in range(LAG):
        fire(b)

    def body(b, _):
        drain_one_batch()
        fire(b)
        return _

    lax.fori_loop(LAG, B, body, None)
    for _ in range(LAG):
        drain_one_batch()


def kernel(input_ids, token_table, pos_table, gamma, beta):
    table, ids_t = pl.pallas_call(
        _table_kernel,
        grid=(VOCAB,),
        in_specs=[
            pl.BlockSpec((B, L), lambda v: (0, 0)),
            pl.BlockSpec((VOCAB, H), lambda v: (0, 0)),
            pl.BlockSpec((1, 1, H), lambda v: (v, 0, 0)),
            pl.BlockSpec((L, H), lambda v: (0, 0)),
            pl.BlockSpec((1, H), lambda v: (0, 0)),
            pl.BlockSpec((1, H), lambda v: (0, 0)),
        ],
        out_specs=(
            pl.BlockSpec((L, 1, 1, H), lambda v: (0, v, 0, 0)),
            pl.BlockSpec((L, B), lambda v: (0, 0)),
        ),
        out_shape=(
            jax.ShapeDtypeStruct((L, VOCAB, 1, H), jnp.float32),
            jax.ShapeDtypeStruct((L, B), jnp.int32),
        ),
        scratch_shapes=[
            pltpu.VMEM((L, VOCAB), jnp.float32),
            pltpu.VMEM((L, VOCAB), jnp.float32),
        ],
        compiler_params=pltpu.CompilerParams(
            dimension_semantics=("arbitrary",),
        ),
    )(input_ids.astype(jnp.int32), token_table,
      token_table.reshape(VOCAB, 1, H), pos_table,
      gamma.reshape(1, H), beta.reshape(1, H))
    table = table.reshape(L * VOCAB, H)
    ids_t = ids_t.reshape(L * B)
    sc_call = functools.partial(
        pl.kernel,
        mesh=plsc.VectorSubcoreMesh(core_axis_name="c", subcore_axis_name="s"),
        compiler_params=pltpu.CompilerParams(needs_layout_passes=False),
        out_type=jax.ShapeDtypeStruct((N, H), jnp.float32),
        scratch_types=[
            pltpu.VMEM((LPW * VOCAB, H), jnp.float32),
            pltpu.VMEM((LPW * B,), jnp.int32),
            pltpu.SemaphoreType.DMA,
        ],
    )(_sc_gather)
    out = sc_call(table, ids_t)
    return out.reshape(B, L, H)


# speedup vs baseline: 1.0890x; 1.0890x over previous
import functools

import jax
import jax.numpy as jnp
from jax import lax
from jax.experimental import pallas as pl
from jax.experimental.pallas import tpu as pltpu
from jax.experimental.pallas import tpu_sc as plsc

B, L, H, VOCAB = 128, 512, 768, 8
EPS = 1e-5

NC, NS, LANES = 2, 16, 16
NW = NC * NS
N = B * L


def _table_kernel(ids_ref, tok_ref, pos_ref, gamma_ref, beta_ref, t_ref, idst_ref):
    idst_ref[...] = ids_ref[...].T
    tok = tok_ref[...]
    pos = pos_ref[...]
    inv_h = 1.0 / H
    ones_row = jnp.ones((1, H), dtype=jnp.float32)
    mp = jnp.mean(pos, axis=1, keepdims=True)
    ep2 = jnp.mean(pos * pos, axis=1, keepdims=True)
    mt = lax.dot_general(ones_row, tok, (((1,), (1,)), ((), ())),
                         preferred_element_type=jnp.float32) * inv_h
    et2 = lax.dot_general(ones_row, tok * tok, (((1,), (1,)), ((), ())),
                          preferred_element_type=jnp.float32) * inv_h
    cross = lax.dot_general(pos, tok, (((1,), (1,)), ((), ())),
                            preferred_element_type=jnp.float32) * inv_h
    mu = mp + mt
    var = ep2 + et2 + 2.0 * cross - mu * mu
    rstd = lax.rsqrt(var + EPS)
    rm = rstd * mu
    gamma = gamma_ref[0]
    beta = beta_ref[0]
    for v in range(VOCAB):
        t = (pos + tok[v, :][None, :]) * rstd[:, v:v + 1] - rm[:, v:v + 1]
        t_ref[:, v, :] = t * gamma[None, :] + beta[None, :]


LPW = L // NW


LAG = 4


def _sc_gather(t_hbm, idst_hbm, out_hbm, tl_v, ids_v, sem):
    wid = lax.axis_index("s") * NC + lax.axis_index("c")
    l0 = wid * LPW
    pltpu.sync_copy(t_hbm.at[pl.ds(wid * LPW * VOCAB, LPW * VOCAB)], tl_v)
    pltpu.sync_copy(idst_hbm.at[pl.ds(l0 * B, LPW * B)], ids_v)

    lanesb = lax.iota(jnp.int32, LANES) * B

    def fire(b):
        v = plsc.load_gather(ids_v, [lanesb + b])
        for l in range(LPW):
            r = v[l] + l * VOCAB
            pltpu.async_copy(
                tl_v.at[pl.ds(r, 1)], out_hbm.at[pl.ds(b * L + l0 + l, 1)], sem
            )

    def drain_one_batch():
        pltpu.make_async_copy(
            tl_v.at[pl.ds(0, LPW)], out_hbm.at[pl.ds(l0, LPW)], sem
        ).wait()

    for b in range(LAG):
        fire(b)

    def body(b, _):
        drain_one_batch()
        fire(b)
        return _

    lax.fori_loop(LAG, B, body, None)
    for _ in range(LAG):
        drain_one_batch()


def kernel(input_ids, token_table, pos_table, gamma, beta):
    table, ids_t = pl.pallas_call(
        _table_kernel,
        out_shape=(
            jax.ShapeDtypeStruct((L, VOCAB, H), jnp.float32),
            jax.ShapeDtypeStruct((L, B), jnp.int32),
        ),
    )(input_ids.astype(jnp.int32), token_table, pos_table,
      gamma.reshape(1, H), beta.reshape(1, H))
    table = table.reshape(L * VOCAB, H)
    ids_t = ids_t.reshape(L * B)
    sc_call = functools.partial(
        pl.kernel,
        mesh=plsc.VectorSubcoreMesh(core_axis_name="c", subcore_axis_name="s"),
        compiler_params=pltpu.CompilerParams(needs_layout_passes=False),
        out_type=jax.ShapeDtypeStruct((N, H), jnp.float32),
        scratch_types=[
            pltpu.VMEM((LPW * VOCAB, H), jnp.float32),
            pltpu.VMEM((LPW * B,), jnp.int32),
            pltpu.SemaphoreType.DMA,
        ],
    )(_sc_gather)
    out = sc_call(table, ids_t)
    return out.reshape(B, L, H)


# speedup vs baseline: 1.1087x; 1.0181x over previous
import functools

import jax
import jax.numpy as jnp
from jax import lax
from jax.experimental import pallas as pl
from jax.experimental.pallas import tpu as pltpu
from jax.experimental.pallas import tpu_sc as plsc

B, L, H, VOCAB = 128, 512, 768, 8
EPS = 1e-5

NC, NS, LANES = 2, 16, 16
NW = NC * NS
N = B * L


LB = L // VOCAB


def _table_kernel(ids_ref, tok_ref, pos_ref, posf_ref, gamma_ref, beta_ref,
                  t_ref, idst_ref, rstd_s, rm_s):
    i = pl.program_id(0)

    @pl.when(i == 0)
    def _():
        idst_ref[...] = ids_ref[...].T
        tok = tok_ref[...]
        pos = posf_ref[...]
        inv_h = 1.0 / H
        ones_row = jnp.ones((1, H), dtype=jnp.float32)
        mp = jnp.mean(pos, axis=1, keepdims=True)
        ep2 = jnp.mean(pos * pos, axis=1, keepdims=True)
        mt = lax.dot_general(ones_row, tok, (((1,), (1,)), ((), ())),
                             preferred_element_type=jnp.float32) * inv_h
        et2 = lax.dot_general(ones_row, tok * tok, (((1,), (1,)), ((), ())),
                              preferred_element_type=jnp.float32) * inv_h
        cross = lax.dot_general(pos, tok, (((1,), (1,)), ((), ())),
                                preferred_element_type=jnp.float32) * inv_h
        mu = mp + mt
        var = ep2 + et2 + 2.0 * cross - mu * mu
        rstd = lax.rsqrt(var + EPS)
        rstd_s[...] = rstd
        rm_s[...] = rstd * mu

    tok = tok_ref[...]
    pos = pos_ref[...]
    rstd = rstd_s[pl.ds(i * LB, LB), :]
    rm = rm_s[pl.ds(i * LB, LB), :]
    gamma = gamma_ref[0]
    beta = beta_ref[0]
    for v in range(VOCAB):
        t = (pos + tok[v, :][None, :]) * rstd[:, v:v + 1] - rm[:, v:v + 1]
        t_ref[:, v, :] = t * gamma[None, :] + beta[None, :]


LPW = L // NW


LAG = 4


def _sc_gather(t_hbm, idst_hbm, out_hbm, tl_v, ids_v, sem):
    wid = lax.axis_index("s") * NC + lax.axis_index("c")
    l0 = wid * LPW
    pltpu.sync_copy(t_hbm.at[pl.ds(wid * LPW * VOCAB, LPW * VOCAB)], tl_v)
    pltpu.sync_copy(idst_hbm.at[pl.ds(l0 * B, LPW * B)], ids_v)

    lanesb = lax.iota(jnp.int32, LANES) * B

    def fire(b):
        v = plsc.load_gather(ids_v, [lanesb + b])
        for l in range(LPW):
            r = v[l] + l * VOCAB
            pltpu.async_copy(
                tl_v.at[pl.ds(r, 1)], out_hbm.at[pl.ds(b * L + l0 + l, 1)], sem
            )

    def drain_one_batch():
        pltpu.make_async_copy(
            tl_v.at[pl.ds(0, LPW)], out_hbm.at[pl.ds(l0, LPW)], sem
        ).wait()

    for b in range(LAG):
        fire(b)

    def body(b, _):
        drain_one_batch()
        fire(b)
        return _

    lax.fori_loop(LAG, B, body, None)
    for _ in range(LAG):
        drain_one_batch()


def kernel(input_ids, token_table, pos_table, gamma, beta):
    table, ids_t = pl.pallas_call(
        _table_kernel,
        grid=(VOCAB,),
        in_specs=[
            pl.BlockSpec((B, L), lambda i: (0, 0)),
            pl.BlockSpec((VOCAB, H), lambda i: (0, 0)),
            pl.BlockSpec((LB, H), lambda i: (i, 0)),
            pl.BlockSpec((L, H), lambda i: (0, 0)),
            pl.BlockSpec((1, H), lambda i: (0, 0)),
            pl.BlockSpec((1, H), lambda i: (0, 0)),
        ],
        out_specs=(
            pl.BlockSpec((LB, VOCAB, H), lambda i: (i, 0, 0)),
            pl.BlockSpec((L, B), lambda i: (0, 0)),
        ),
        out_shape=(
            jax.ShapeDtypeStruct((L, VOCAB, H), jnp.float32),
            jax.ShapeDtypeStruct((L, B), jnp.int32),
        ),
        scratch_shapes=[
            pltpu.VMEM((L, VOCAB), jnp.float32),
            pltpu.VMEM((L, VOCAB), jnp.float32),
        ],
        compiler_params=pltpu.CompilerParams(
            dimension_semantics=("arbitrary",),
        ),
    )(input_ids.astype(jnp.int32), token_table, pos_table, pos_table,
      gamma.reshape(1, H), beta.reshape(1, H))
    table = table.reshape(L * VOCAB, H)
    ids_t = ids_t.reshape(L * B)
    sc_call = functools.partial(
        pl.kernel,
        mesh=plsc.VectorSubcoreMesh(core_axis_name="c", subcore_axis_name="s"),
        compiler_params=pltpu.CompilerParams(needs_layout_passes=False),
        out_type=jax.ShapeDtypeStruct((N, H), jnp.float32),
        scratch_types=[
            pltpu.VMEM((LPW * VOCAB, H), jnp.float32),
            pltpu.VMEM((LPW * B,), jnp.int32),
            pltpu.SemaphoreType.DMA,
        ],
    )(_sc_gather)
    out = sc_call(table, ids_t)
    return out.reshape(B, L, H)
